# baseline (speedup 1.0000x reference)
import functools

import jax
import jax.numpy as jnp
from jax import lax
from jax.experimental import pallas as pl
from jax.experimental.pallas import tpu as pltpu
from jax.experimental.pallas import tpu_sc as plsc

B, L, V, D = 4096, 200, 100000, 64
N = B * L
NC, NS = 2, 16
NW = NC * NS
ROWS_PER_W = N // NW
CK = L
NCH = ROWS_PER_W // CK
LANES = 16


def _sc_embed(x_flat, token_weight, pos_weight):
    mesh = plsc.VectorSubcoreMesh(core_axis_name="c", subcore_axis_name="s")

    @functools.partial(
        pl.kernel,
        mesh=mesh,
        compiler_params=pltpu.CompilerParams(use_tc_tiling_on_sc=False),
        out_type=jax.ShapeDtypeStruct((N, D), jnp.float32),
        scratch_types=(
            [pltpu.VMEM((CK,), jnp.int32) for _ in range(2)]
            + [pltpu.VMEM((CK, D), jnp.float32) for _ in range(2)]
            + [pltpu.VMEM((L, D), jnp.float32)]
            + [pltpu.SemaphoreType.DMA for _ in range(6)]
        ),
    )
    def k(x_hbm, tok_hbm, pos_hbm, out_hbm, *s):
        idx_b = s[0:2]
        rows_b = s[2:4]
        pos_v = s[4]
        si = s[5:7]
        sg = s[7:9]
        so = s[9:11]

        wid = lax.axis_index("s") * NC + lax.axis_index("c")
        base = wid * ROWS_PER_W
        pltpu.sync_copy(pos_hbm, pos_v)

        def fire_idx(c, b):
            pltpu.async_copy(x_hbm.at[pl.ds(base + c * CK, CK)], idx_b[b], si[b])

        def wait_idx(b):
            pltpu.make_async_copy(x_hbm.at[pl.ds(0, CK)], idx_b[b], si[b]).wait()

        def fire_gather(b):
            pltpu.async_copy(tok_hbm.at[idx_b[b]], rows_b[b], sg[b])

        def wait_gather(b):
            pltpu.make_async_copy(tok_hbm.at[pl.ds(0, CK)], rows_b[b],
                                  sg[b]).wait()

        def fire_out(c, b):
            pltpu.async_copy(rows_b[b], out_hbm.at[pl.ds(base + c * CK, CK)],
                             so[b])

        def wait_out(b):
            pltpu.make_async_copy(out_hbm.at[pl.ds(0, CK)], rows_b[b],
                                  so[b]).wait()

        def add_pos(b):
            rows = rows_b[b]

            def row_body(r, carry):
                for rep in range(CK // L):
                    row = rep * L + r
                    for kk in range(D // LANES):
                        sl = pl.ds(kk * LANES, LANES)
                        rows[row, sl] = rows[row, sl] + pos_v[r, sl]
                return carry

            lax.fori_loop(0, L, row_body, 0)

        fire_idx(0, 0)
        wait_idx(0)
        fire_gather(0)
        fire_idx(1, 1)

        def pair_body(p, carry):
            for b in range(2):
                c = 2 * p + b
                nb = 1 - b
                wait_gather(b)

                @pl.when(c + 1 < NCH)
                def _():
                    wait_idx(nb)

                    @pl.when(c >= 1)
                    def _():
                        wait_out(nb)

                    fire_gather(nb)

                    @pl.when(c + 2 < NCH)
                    def _():
                        fire_idx(c + 2, b)

                add_pos(b)
                fire_out(c, b)
            return carry

        lax.fori_loop(0, NCH // 2, pair_body, 0)
        wait_out(0)
        wait_out(1)

    return k(x_flat, token_weight, pos_weight)


def kernel(x, token_weight, pos_weight):
    x_flat = x.reshape(-1).astype(jnp.int32)
    out = _sc_embed(x_flat, token_weight, pos_weight)
    return out.reshape(B, L, D)


# speedup vs baseline: 1.0393x; 1.0393x over previous
import functools

import jax
import jax.numpy as jnp
from jax import lax
from jax.experimental import pallas as pl
from jax.experimental.pallas import tpu as pltpu
from jax.experimental.pallas import tpu_sc as plsc

B, L, V, D = 4096, 200, 100000, 64
N = B * L
NC, NS = 2, 16
NW = NC * NS
ROWS_PER_W = N // NW
CK = 2 * L
CH = CK // 2
NCH = ROWS_PER_W // CK
LANES = 16


def _sc_embed(x_flat, token_weight, pos_weight):
    mesh = plsc.VectorSubcoreMesh(core_axis_name="c", subcore_axis_name="s")

    @functools.partial(
        pl.kernel,
        mesh=mesh,
        compiler_params=pltpu.CompilerParams(use_tc_tiling_on_sc=False),
        out_type=jax.ShapeDtypeStruct((N, D), jnp.float32),
        scratch_types=(
            [pltpu.VMEM((CH,), jnp.int32) for _ in range(4)]
            + [pltpu.VMEM((CH, D), jnp.float32) for _ in range(4)]
            + [pltpu.VMEM((L, D), jnp.float32)]
            + [pltpu.SemaphoreType.DMA for _ in range(12)]
        ),
    )
    def k(x_hbm, tok_hbm, pos_hbm, out_hbm, *s):
        idx_b = ((s[0], s[1]), (s[2], s[3]))
        rows_b = ((s[4], s[5]), (s[6], s[7]))
        pos_v = s[8]
        si = ((s[9], s[10]), (s[11], s[12]))
        sg = ((s[13], s[14]), (s[15], s[16]))
        so = ((s[17], s[18]), (s[19], s[20]))

        wid = lax.axis_index("s") * NC + lax.axis_index("c")
        base = wid * ROWS_PER_W
        pltpu.sync_copy(pos_hbm, pos_v)

        def fire_idx(c, b):
            for q in range(2):
                pltpu.async_copy(x_hbm.at[pl.ds(base + c * CK + q * CH, CH)],
                                 idx_b[b][q], si[b][q])

        def wait_idx(b):
            for q in range(2):
                pltpu.make_async_copy(x_hbm.at[pl.ds(0, CH)], idx_b[b][q],
                                      si[b][q]).wait()

        def fire_gather(b):
            for q in range(2):
                pltpu.async_copy(tok_hbm.at[idx_b[b][q]], rows_b[b][q],
                                 sg[b][q])

        def wait_gather(b):
            for q in range(2):
                pltpu.make_async_copy(tok_hbm.at[pl.ds(0, CH)], rows_b[b][q],
                                      sg[b][q]).wait()

        def fire_out(c, b):
            for q in range(2):
                pltpu.async_copy(rows_b[b][q],
                                 out_hbm.at[pl.ds(base + c * CK + q * CH, CH)],
                                 so[b][q])

        def wait_out(b):
            for q in range(2):
                pltpu.make_async_copy(out_hbm.at[pl.ds(0, CH)], rows_b[b][q],
                                      so[b][q]).wait()

        def add_pos(b):
            def row_body(r, carry):
                for q in range(2):
                    rows = rows_b[b][q]
                    for kk in range(D // LANES):
                        sl = pl.ds(kk * LANES, LANES)
                        rows[r, sl] = rows[r, sl] + pos_v[r, sl]
                return carry

            lax.fori_loop(0, L, row_body, 0)

        fire_idx(0, 0)
        wait_idx(0)
        fire_gather(0)
        fire_idx(1, 1)

        def pair_body(p, carry):
            for b in range(2):
                c = 2 * p + b
                nb = 1 - b
                wait_gather(b)

                @pl.when(c + 1 < NCH)
                def _():
                    wait_idx(nb)

                    @pl.when(c >= 1)
                    def _():
                        wait_out(nb)

                    fire_gather(nb)

                    @pl.when(c + 2 < NCH)
                    def _():
                        fire_idx(c + 2, b)

                add_pos(b)
                fire_out(c, b)
            return carry

        lax.fori_loop(0, NCH // 2, pair_body, 0)
        wait_out(0)
        wait_out(1)

    return k(x_flat, token_weight, pos_weight)


def kernel(x, token_weight, pos_weight):
    x_flat = x.reshape(-1).astype(jnp.int32)
    out = _sc_embed(x_flat, token_weight, pos_weight)
    return out.reshape(B, L, D)
